# Initial kernel scaffold; baseline (speedup 1.0000x reference)
import jax
import jax.numpy as jnp
from jax.experimental import pallas as pl


def kernel(X, label_embeds, edge_index, W1, b1, W2, b2, W3, b3):
    raise NotImplementedError("write your pallas kernel here")


# speedup vs baseline: 8.0515x; 8.0515x over previous
import functools

import jax
import jax.numpy as jnp
from jax import lax
from jax.experimental import pallas as pl
from jax.experimental.pallas import tpu as pltpu
from jax.experimental.pallas import tpu_sc as plsc

N = 10000
E = 320000
B = 1024
D_EMB = 128

NC = 2
NS = 16
NW = NC * NS
EPT = E // NW
K = 80
NCH = EPT // K

_f32 = jnp.float32


def _mesh():
    return plsc.VectorSubcoreMesh(core_axis_name="c", subcore_axis_name="s")


def _flush(acc, out_hbm, c, s):
    r0 = 624
    last = N - (NS - 1) * r0

    @pl.when(s < NS - 1)
    def _():
        off = pl.multiple_of(s * r0, 8)
        pltpu.sync_copy(acc.at[pl.ds(off, r0)], out_hbm.at[c, pl.ds(off, r0)])

    @pl.when(s == NS - 1)
    def _():
        off = (NS - 1) * r0
        pltpu.sync_copy(acc.at[pl.ds(off, last)],
                        out_hbm.at[c, pl.ds(off, last)])


def _make_deg_kernel(w):
    @functools.partial(
        pl.kernel,
        out_type=jax.ShapeDtypeStruct((NC, 2, N, w), _f32),
        mesh=_mesh(),
        scratch_types=[
            pltpu.VMEM((NCH, K), jnp.int32),
            pltpu.VMEM((NCH, K), jnp.int32),
            pltpu.VMEM((K, w), _f32),
            pltpu.VMEM_SHARED((N, w), _f32),
            pltpu.VMEM_SHARED((N, w), _f32),
        ],
        compiler_params=pltpu.CompilerParams(use_tc_tiling_on_sc=False),
    )
    def deg(src_hbm, dst_hbm, z_hbm, pat_hbm, out_hbm,
            idx_s, idx_d, ones, acc_s, acc_d):
        c = lax.axis_index("c")
        s = lax.axis_index("s")
        wid = c * NS + s

        @pl.when(s == 0)
        def _():
            pltpu.sync_copy(z_hbm, acc_s)

        @pl.when(s == 1)
        def _():
            pltpu.sync_copy(z_hbm, acc_d)

        pltpu.sync_copy(pat_hbm, ones)
        pltpu.sync_copy(src_hbm.at[wid], idx_s)
        pltpu.sync_copy(dst_hbm.at[wid], idx_d)
        plsc.subcore_barrier()

        def step_s(j, carry):
            pltpu.sync_copy(ones, acc_s.at[idx_s.at[j]], add=True)
            return carry

        lax.fori_loop(0, NCH, step_s, 0)

        def step_d(j, carry):
            pltpu.sync_copy(ones, acc_d.at[idx_d.at[j]], add=True)
            return carry

        lax.fori_loop(0, NCH, step_d, 0)
        plsc.subcore_barrier()
        _flush(acc_s, out_hbm.at[c], 0, s)
        _flush(acc_d, out_hbm.at[c], 1, s)

    return deg


_DEG_W = 8
_deg_kernel = _make_deg_kernel(_DEG_W)


def _make_mp_kernel(d):
    @functools.partial(
        pl.kernel,
        out_type=jax.ShapeDtypeStruct((NC, N, d), _f32),
        mesh=_mesh(),
        scratch_types=[
            pltpu.VMEM((NCH, K), jnp.int32),
            pltpu.VMEM((NCH, K), jnp.int32),
            pltpu.VMEM((K, d), _f32),
            pltpu.VMEM_SHARED((N, d), _f32),
            pltpu.SemaphoreType.DMA,
        ],
        compiler_params=pltpu.CompilerParams(use_tc_tiling_on_sc=False),
    )
    def mp(y_hbm, src_hbm, dst_hbm, z_hbm, out_hbm, idx_s, idx_d, buf, acc, sem):
        c = lax.axis_index("c")
        s = lax.axis_index("s")
        wid = c * NS + s

        @pl.when(s == 0)
        def _():
            pltpu.sync_copy(z_hbm, acc)

        pltpu.sync_copy(src_hbm.at[wid], idx_s)
        pltpu.sync_copy(dst_hbm.at[wid], idx_d)
        plsc.subcore_barrier()

        def step(j, carry):
            pltpu.async_copy(y_hbm.at[idx_s.at[j]], buf, sem).wait()
            pltpu.sync_copy(buf, acc.at[idx_d.at[j]], add=True)
            return carry

        lax.fori_loop(0, NCH, step, 0)
        plsc.subcore_barrier()
        _flush(acc, out_hbm, c, s)

    return mp


_mp32 = _make_mp_kernel(32)
_mp16 = _make_mp_kernel(16)


def _tc1_body(ds0, ds1, dd0, dd1, emb, w1, rso_ref, rsi_ref, y1_ref):
    rso = lax.rsqrt(jnp.maximum((ds0[...] + ds1[...])[:, 0:1], 1.0))
    rsi = lax.rsqrt(jnp.maximum((dd0[...] + dd1[...])[:, 0:1], 1.0))
    rso_ref[...] = rso
    rsi_ref[...] = rsi
    y1_ref[...] = jnp.dot(emb[...] * rso, w1[...],
                          preferred_element_type=_f32)


def _tc2_body(p0, p1, rsi, rso, b1, w2, y2_ref):
    h = jnp.maximum((p0[...] + p1[...]) * rsi[...] + b1[...], 0.0)
    y2_ref[...] = jnp.dot(h * rso[...], w2[...], preferred_element_type=_f32)


def _tc3_body(p0, p1, rsi, rso, b2, y3_ref):
    h = jnp.maximum((p0[...] + p1[...]) * rsi[...] + b2[...], 0.0)
    y3_ref[...] = h * rso[...]


def _tc4_body(p0, p1, rsi, b3, w3, x, out_ref):
    agg = (p0[...] + p1[...]) * rsi[...]
    h = jnp.maximum(jnp.dot(agg, w3[...], preferred_element_type=_f32)
                    + b3[...], 0.0)
    z = lax.dot_general(x[...], h, (((1,), (1,)), ((), ())),
                        preferred_element_type=_f32)
    out_ref[...] = jax.nn.sigmoid(z)


def kernel(X, label_embeds, edge_index, W1, b1, W2, b2, W3, b3):
    src = edge_index[0].reshape(NW, NCH, K)
    dst = edge_index[1].reshape(NW, NCH, K)

    z32 = jnp.zeros((N, 32), _f32)
    z16 = jnp.zeros((N, 16), _f32)
    zw = jnp.zeros((N, _DEG_W), _f32)
    pat = jnp.ones((K, _DEG_W), _f32)

    dp = _deg_kernel(src, dst, zw, pat)

    rso, rsi, y1 = pl.pallas_call(
        _tc1_body,
        out_shape=(jax.ShapeDtypeStruct((N, 1), _f32),
                   jax.ShapeDtypeStruct((N, 1), _f32),
                   jax.ShapeDtypeStruct((N, 32), _f32)),
    )(dp[0, 0], dp[1, 0], dp[0, 1], dp[1, 1], label_embeds, W1)

    p = _mp32(y1, src, dst, z32)

    y2 = pl.pallas_call(
        _tc2_body, out_shape=jax.ShapeDtypeStruct((N, 16), _f32),
    )(p[0], p[1], rsi, rso, b1.reshape(1, 32), W2)

    p = _mp16(y2, src, dst, z16)

    y3 = pl.pallas_call(
        _tc3_body, out_shape=jax.ShapeDtypeStruct((N, 16), _f32),
    )(p[0], p[1], rsi, rso, b2.reshape(1, 16))

    p = _mp16(y3, src, dst, z16)

    out = pl.pallas_call(
        _tc4_body, out_shape=jax.ShapeDtypeStruct((B, N), _f32),
    )(p[0], p[1], rsi, b3.reshape(1, D_EMB), W3, X)
    return out


# speedup vs baseline: 11.8364x; 1.4701x over previous
import functools

import jax
import jax.numpy as jnp
from jax import lax
from jax.experimental import pallas as pl
from jax.experimental.pallas import tpu as pltpu
from jax.experimental.pallas import tpu_sc as plsc

N = 10000
E = 320000
B = 1024
D_EMB = 128

NC = 2
NS = 16
NW = NC * NS
EPT = E // NW
K = 80
NCH = EPT // K

_f32 = jnp.float32


def _mesh():
    return plsc.VectorSubcoreMesh(core_axis_name="c", subcore_axis_name="s")


def _flush(acc, out_hbm, c, s):
    r0 = 624
    last = N - (NS - 1) * r0

    @pl.when(s < NS - 1)
    def _():
        off = pl.multiple_of(s * r0, 8)
        pltpu.sync_copy(acc.at[pl.ds(off, r0)], out_hbm.at[c, pl.ds(off, r0)])

    @pl.when(s == NS - 1)
    def _():
        off = (NS - 1) * r0
        pltpu.sync_copy(acc.at[pl.ds(off, last)],
                        out_hbm.at[c, pl.ds(off, last)])


def _make_deg_kernel(w):
    @functools.partial(
        pl.kernel,
        out_type=jax.ShapeDtypeStruct((NC, 2, N, w), _f32),
        mesh=_mesh(),
        scratch_types=[
            pltpu.VMEM((NCH, K), jnp.int32),
            pltpu.VMEM((NCH, K), jnp.int32),
            pltpu.VMEM((K, w), _f32),
            pltpu.VMEM_SHARED((N, w), _f32),
            pltpu.VMEM_SHARED((N, w), _f32),
        ],
        compiler_params=pltpu.CompilerParams(use_tc_tiling_on_sc=False),
    )
    def deg(src_hbm, dst_hbm, z_hbm, pat_hbm, out_hbm,
            idx_s, idx_d, ones, acc_s, acc_d):
        c = lax.axis_index("c")
        s = lax.axis_index("s")
        wid = c * NS + s

        @pl.when(s == 0)
        def _():
            pltpu.sync_copy(z_hbm, acc_s)

        @pl.when(s == 1)
        def _():
            pltpu.sync_copy(z_hbm, acc_d)

        pltpu.sync_copy(pat_hbm, ones)
        pltpu.sync_copy(src_hbm.at[wid], idx_s)
        pltpu.sync_copy(dst_hbm.at[wid], idx_d)
        plsc.subcore_barrier()

        def step_s(j, carry):
            pltpu.sync_copy(ones, acc_s.at[idx_s.at[j]], add=True)
            return carry

        lax.fori_loop(0, NCH, step_s, 0)

        def step_d(j, carry):
            pltpu.sync_copy(ones, acc_d.at[idx_d.at[j]], add=True)
            return carry

        lax.fori_loop(0, NCH, step_d, 0)
        plsc.subcore_barrier()
        _flush(acc_s, out_hbm.at[c], 0, s)
        _flush(acc_d, out_hbm.at[c], 1, s)

    return deg


_DEG_W = 8
_deg_kernel = _make_deg_kernel(_DEG_W)


def _make_mp_kernel(d):
    @functools.partial(
        pl.kernel,
        out_type=jax.ShapeDtypeStruct((NC, N, d), _f32),
        mesh=_mesh(),
        scratch_types=[
            pltpu.VMEM((NCH, K), jnp.int32),
            pltpu.VMEM((NCH, K), jnp.int32),
            pltpu.VMEM((K, d), _f32),
            pltpu.VMEM_SHARED((N, d), _f32),
            pltpu.VMEM_SHARED((N, d), _f32),
            pltpu.SemaphoreType.DMA,
        ],
        compiler_params=pltpu.CompilerParams(use_tc_tiling_on_sc=False),
    )
    def mp(y_hbm, src_hbm, dst_hbm, z_hbm, out_hbm, idx_s, idx_d, buf, y_sh,
           acc, sem):
        c = lax.axis_index("c")
        s = lax.axis_index("s")
        wid = c * NS + s

        @pl.when(s == 0)
        def _():
            pltpu.sync_copy(z_hbm, acc)

        rows = N // NS
        pltpu.sync_copy(y_hbm.at[pl.ds(s * rows, rows)],
                        y_sh.at[pl.ds(s * rows, rows)])
        pltpu.sync_copy(src_hbm.at[wid], idx_s)
        pltpu.sync_copy(dst_hbm.at[wid], idx_d)
        plsc.subcore_barrier()

        def step(j, carry):
            pltpu.async_copy(y_sh.at[idx_s.at[j]], buf, sem).wait()
            pltpu.sync_copy(buf, acc.at[idx_d.at[j]], add=True)
            return carry

        lax.fori_loop(0, NCH, step, 0)
        plsc.subcore_barrier()
        _flush(acc, out_hbm, c, s)

    return mp


_mp32 = _make_mp_kernel(32)
_mp16 = _make_mp_kernel(16)


def _tc1_body(ds0, ds1, dd0, dd1, emb, w1, rso_ref, rsi_ref, y1_ref):
    rso = lax.rsqrt(jnp.maximum((ds0[...] + ds1[...])[:, 0:1], 1.0))
    rsi = lax.rsqrt(jnp.maximum((dd0[...] + dd1[...])[:, 0:1], 1.0))
    rso_ref[...] = rso
    rsi_ref[...] = rsi
    y1_ref[...] = jnp.dot(emb[...] * rso, w1[...],
                          preferred_element_type=_f32)


def _tc2_body(p0, p1, rsi, rso, b1, w2, y2_ref):
    h = jnp.maximum((p0[...] + p1[...]) * rsi[...] + b1[...], 0.0)
    y2_ref[...] = jnp.dot(h * rso[...], w2[...], preferred_element_type=_f32)


def _tc3_body(p0, p1, rsi, rso, b2, y3_ref):
    h = jnp.maximum((p0[...] + p1[...]) * rsi[...] + b2[...], 0.0)
    y3_ref[...] = h * rso[...]


def _tc4_body(p0, p1, rsi, b3, w3, x, out_ref):
    agg = (p0[...] + p1[...]) * rsi[...]
    h = jnp.maximum(jnp.dot(agg, w3[...], preferred_element_type=_f32)
                    + b3[...], 0.0)
    z = lax.dot_general(x[...], h, (((1,), (1,)), ((), ())),
                        preferred_element_type=_f32)
    out_ref[...] = jax.nn.sigmoid(z)


def kernel(X, label_embeds, edge_index, W1, b1, W2, b2, W3, b3):
    src = edge_index[0].reshape(NW, NCH, K)
    dst = edge_index[1].reshape(NW, NCH, K)

    z32 = jnp.zeros((N, 32), _f32)
    z16 = jnp.zeros((N, 16), _f32)
    zw = jnp.zeros((N, _DEG_W), _f32)
    pat = jnp.ones((K, _DEG_W), _f32)

    dp = _deg_kernel(src, dst, zw, pat)

    rso, rsi, y1 = pl.pallas_call(
        _tc1_body,
        out_shape=(jax.ShapeDtypeStruct((N, 1), _f32),
                   jax.ShapeDtypeStruct((N, 1), _f32),
                   jax.ShapeDtypeStruct((N, 32), _f32)),
    )(dp[0, 0], dp[1, 0], dp[0, 1], dp[1, 1], label_embeds, W1)

    p = _mp32(y1, src, dst, z32)

    y2 = pl.pallas_call(
        _tc2_body, out_shape=jax.ShapeDtypeStruct((N, 16), _f32),
    )(p[0], p[1], rsi, rso, b1.reshape(1, 32), W2)

    p = _mp16(y2, src, dst, z16)

    y3 = pl.pallas_call(
        _tc3_body, out_shape=jax.ShapeDtypeStruct((N, 16), _f32),
    )(p[0], p[1], rsi, rso, b2.reshape(1, 16))

    p = _mp16(y3, src, dst, z16)

    out = pl.pallas_call(
        _tc4_body, out_shape=jax.ShapeDtypeStruct((B, N), _f32),
    )(p[0], p[1], rsi, b3.reshape(1, D_EMB), W3, X)
    return out


# speedup vs baseline: 13.4440x; 1.1358x over previous
import functools

import jax
import jax.numpy as jnp
from jax import lax
from jax.experimental import pallas as pl
from jax.experimental.pallas import tpu as pltpu
from jax.experimental.pallas import tpu_sc as plsc

N = 10000
E = 320000
B = 1024
D_EMB = 128

NC = 2
NS = 16
NW = NC * NS
EPT = E // NW
K = 80
NCH = EPT // K

_f32 = jnp.float32


def _mesh():
    return plsc.VectorSubcoreMesh(core_axis_name="c", subcore_axis_name="s")


def _flush(acc, out_hbm, c, s):
    r0 = 624
    last = N - (NS - 1) * r0

    @pl.when(s < NS - 1)
    def _():
        off = pl.multiple_of(s * r0, 8)
        pltpu.sync_copy(acc.at[pl.ds(off, r0)], out_hbm.at[c, pl.ds(off, r0)])

    @pl.when(s == NS - 1)
    def _():
        off = (NS - 1) * r0
        pltpu.sync_copy(acc.at[pl.ds(off, last)],
                        out_hbm.at[c, pl.ds(off, last)])


def _make_deg_kernel(w):
    @functools.partial(
        pl.kernel,
        out_type=jax.ShapeDtypeStruct((NC, 2, N, w), _f32),
        mesh=_mesh(),
        scratch_types=[
            pltpu.VMEM((NCH, K), jnp.int32),
            pltpu.VMEM((NCH, K), jnp.int32),
            pltpu.VMEM((K, w), _f32),
            pltpu.VMEM_SHARED((N, w), _f32),
            pltpu.VMEM_SHARED((N, w), _f32),
        ],
        compiler_params=pltpu.CompilerParams(use_tc_tiling_on_sc=False),
    )
    def deg(src_hbm, dst_hbm, z_hbm, pat_hbm, out_hbm,
            idx_s, idx_d, ones, acc_s, acc_d):
        c = lax.axis_index("c")
        s = lax.axis_index("s")
        wid = c * NS + s

        @pl.when(s == 0)
        def _():
            pltpu.sync_copy(z_hbm, acc_s)

        @pl.when(s == 1)
        def _():
            pltpu.sync_copy(z_hbm, acc_d)

        pltpu.sync_copy(pat_hbm, ones)
        pltpu.sync_copy(src_hbm.at[wid], idx_s)
        pltpu.sync_copy(dst_hbm.at[wid], idx_d)
        plsc.subcore_barrier()

        def step_s(j, carry):
            pltpu.sync_copy(ones, acc_s.at[idx_s.at[j]], add=True)
            return carry

        lax.fori_loop(0, NCH, step_s, 0)

        def step_d(j, carry):
            pltpu.sync_copy(ones, acc_d.at[idx_d.at[j]], add=True)
            return carry

        lax.fori_loop(0, NCH, step_d, 0)
        plsc.subcore_barrier()
        _flush(acc_s, out_hbm.at[c], 0, s)
        _flush(acc_d, out_hbm.at[c], 1, s)

    return deg


_DEG_W = 8
_deg_kernel = _make_deg_kernel(_DEG_W)


_G = 5


def _make_mp_kernel(d):
    @functools.partial(
        pl.kernel,
        out_type=jax.ShapeDtypeStruct((NC, N, d), _f32),
        mesh=_mesh(),
        scratch_types=(
            [pltpu.VMEM((NCH, K), jnp.int32)] * 2
            + [pltpu.VMEM((K, d), _f32)] * _G
            + [pltpu.VMEM_SHARED((N, d), _f32)] * 2
            + [pltpu.SemaphoreType.DMA] * (2 * _G)
        ),
        compiler_params=pltpu.CompilerParams(use_tc_tiling_on_sc=False),
    )
    def mp(y_hbm, src_hbm, dst_hbm, z_hbm, out_hbm, idx_s, idx_d,
           b0, b1, b2, b3, b4, y_sh, acc,
           g0, g1, g2, g3, g4, s0, s1, s2, s3, s4):
        bufs = (b0, b1, b2, b3, b4)
        gsem = (g0, g1, g2, g3, g4)
        ssem = (s0, s1, s2, s3, s4)
        c = lax.axis_index("c")
        s = lax.axis_index("s")
        wid = c * NS + s

        @pl.when(s == 0)
        def _():
            pltpu.sync_copy(z_hbm, acc)

        rows = N // NS
        pltpu.sync_copy(y_hbm.at[pl.ds(s * rows, rows)],
                        y_sh.at[pl.ds(s * rows, rows)])
        pltpu.sync_copy(src_hbm.at[wid], idx_s)
        pltpu.sync_copy(dst_hbm.at[wid], idx_d)
        plsc.subcore_barrier()

        def _wait_gather(b):
            pltpu.make_async_copy(y_sh.at[idx_s.at[0]], bufs[b],
                                  gsem[b]).wait()

        def _wait_scatter(b):
            pltpu.make_async_copy(bufs[b], acc.at[idx_d.at[0]],
                                  ssem[b]).wait()

        for b in range(_G - 1):
            pltpu.async_copy(y_sh.at[idx_s.at[b]], bufs[b], gsem[b])

        def group(t, carry):
            for b in range(_G):
                bm = (b + _G - 1) % _G
                m = _G * t + b + (_G - 1)
                if b == 0:
                    @pl.when(t > 0)
                    def _():
                        _wait_scatter(bm)

                    pltpu.async_copy(y_sh.at[idx_s.at[m]], bufs[bm], gsem[bm])
                else:
                    _wait_scatter(bm)

                    @pl.when(m < NCH)
                    def _():
                        pltpu.async_copy(y_sh.at[idx_s.at[m]], bufs[bm],
                                         gsem[bm])

                j = _G * t + b
                _wait_gather(b)
                pltpu.async_copy(bufs[b], acc.at[idx_d.at[j]], ssem[b],
                                 add=True)
            return carry

        lax.fori_loop(0, NCH // _G, group, 0)
        _wait_scatter(_G - 1)
        plsc.subcore_barrier()
        _flush(acc, out_hbm, c, s)

    return mp


_mp32 = _make_mp_kernel(32)
_mp16 = _make_mp_kernel(16)


def _tc1_body(ds0, ds1, dd0, dd1, emb, w1, rso_ref, rsi_ref, y1_ref):
    rso = lax.rsqrt(jnp.maximum((ds0[...] + ds1[...])[:, 0:1], 1.0))
    rsi = lax.rsqrt(jnp.maximum((dd0[...] + dd1[...])[:, 0:1], 1.0))
    rso_ref[...] = rso
    rsi_ref[...] = rsi
    y1_ref[...] = jnp.dot(emb[...] * rso, w1[...],
                          preferred_element_type=_f32)


def _tc2_body(p0, p1, rsi, rso, b1, w2, y2_ref):
    h = jnp.maximum((p0[...] + p1[...]) * rsi[...] + b1[...], 0.0)
    y2_ref[...] = jnp.dot(h * rso[...], w2[...], preferred_element_type=_f32)


def _tc3_body(p0, p1, rsi, rso, b2, y3_ref):
    h = jnp.maximum((p0[...] + p1[...]) * rsi[...] + b2[...], 0.0)
    y3_ref[...] = h * rso[...]


def _tc4_body(p0, p1, rsi, b3, w3, x, out_ref):
    agg = (p0[...] + p1[...]) * rsi[...]
    h = jnp.maximum(jnp.dot(agg, w3[...], preferred_element_type=_f32)
                    + b3[...], 0.0)
    z = lax.dot_general(x[...], h, (((1,), (1,)), ((), ())),
                        preferred_element_type=_f32)
    out_ref[...] = jax.nn.sigmoid(z)


def kernel(X, label_embeds, edge_index, W1, b1, W2, b2, W3, b3):
    src = edge_index[0].reshape(NW, NCH, K)
    dst = edge_index[1].reshape(NW, NCH, K)

    z32 = jnp.zeros((N, 32), _f32)
    z16 = jnp.zeros((N, 16), _f32)
    zw = jnp.zeros((N, _DEG_W), _f32)
    pat = jnp.ones((K, _DEG_W), _f32)

    dp = _deg_kernel(src, dst, zw, pat)

    rso, rsi, y1 = pl.pallas_call(
        _tc1_body,
        out_shape=(jax.ShapeDtypeStruct((N, 1), _f32),
                   jax.ShapeDtypeStruct((N, 1), _f32),
                   jax.ShapeDtypeStruct((N, 32), _f32)),
    )(dp[0, 0], dp[1, 0], dp[0, 1], dp[1, 1], label_embeds, W1)

    p = _mp32(y1, src, dst, z32)

    y2 = pl.pallas_call(
        _tc2_body, out_shape=jax.ShapeDtypeStruct((N, 16), _f32),
    )(p[0], p[1], rsi, rso, b1.reshape(1, 32), W2)

    p = _mp16(y2, src, dst, z16)

    y3 = pl.pallas_call(
        _tc3_body, out_shape=jax.ShapeDtypeStruct((N, 16), _f32),
    )(p[0], p[1], rsi, rso, b2.reshape(1, 16))

    p = _mp16(y3, src, dst, z16)

    out = pl.pallas_call(
        _tc4_body, out_shape=jax.ShapeDtypeStruct((B, N), _f32),
    )(p[0], p[1], rsi, b3.reshape(1, D_EMB), W3, X)
    return out
